# baseline (speedup 1.0000x reference)
import functools

import jax
import jax.numpy as jnp
from jax import lax
from jax.experimental import pallas as pl
from jax.experimental.pallas import tpu as pltpu
from jax.experimental.pallas import tpu_sc as plsc

N = 10000
E = 320000
D = 128

NC = 2
NS = 16
NW = NC * NS
K = 128
CHB = 16
NCK = 5
NB = NCK * CHB
EPT = NB * K
EP = NW * EPT
TR = 8
CH = 1000
NCH = N // CH

_mesh = plsc.VectorSubcoreMesh(core_axis_name="c", subcore_axis_name="s")


NP = 10240
HR = NP // D


@functools.partial(
    pl.kernel,
    out_type=jax.ShapeDtypeStruct((NC, HR, D), jnp.float32),
    mesh=_mesh,
    compiler_params=pltpu.CompilerParams(needs_layout_passes=False),
    scratch_types=[
        pltpu.VMEM((CHB, K), jnp.int32),
        pltpu.VMEM((HR, D), jnp.float32),
        pltpu.VMEM((1, HR), jnp.int32),
        pltpu.VMEM_SHARED((HR, D), jnp.float32),
    ],
)
def _deg_kernel(col_hbm, ridx_hbm, zrows_hbm, out_hbm, colb, hist, ridx, dacc):
    c = lax.axis_index("c")
    s = lax.axis_index("s")
    wid = s * NC + c

    @pl.when(s == 0)
    def _init():
        pltpu.sync_copy(zrows_hbm, dacc)

    pltpu.sync_copy(zrows_hbm, hist)
    pltpu.sync_copy(ridx_hbm, ridx)
    ones16 = jnp.ones((16,), jnp.float32)

    for ck in range(NCK):
        pltpu.sync_copy(col_hbm.at[wid, ck], colb)

        def body(t, carry):
            r = t // 8
            kk = t % 8
            idx = colb[r, pl.ds(kk * 16, 16)]
            rr = lax.shift_right_logical(idx, 7)
            qq = lax.bitwise_and(idx, 127)
            plsc.addupdate_scatter(hist, [rr, qq], ones16)
            return carry

        lax.fori_loop(0, CHB * K // 16, body, 0)
    plsc.subcore_barrier()
    pltpu.sync_copy(hist, dacc.at[ridx.at[0]], add=True)
    plsc.subcore_barrier()

    @pl.when(s == 0)
    def _drain():
        pltpu.sync_copy(dacc, out_hbm.at[c])


@functools.partial(
    pl.kernel,
    out_type=jax.ShapeDtypeStruct((NC, N, D), jnp.float32),
    mesh=_mesh,
    scratch_types=[
        pltpu.VMEM((CHB, K), jnp.int32),
        pltpu.VMEM((CHB, K), jnp.int32),
        pltpu.VMEM((K, D), jnp.float32),
        pltpu.VMEM((K, D), jnp.float32),
        pltpu.VMEM_SHARED((N + TR, D), jnp.float32),
        pltpu.SemaphoreType.DMA,
        pltpu.SemaphoreType.DMA,
        pltpu.SemaphoreType.DMA,
        pltpu.SemaphoreType.DMA,
    ],
)
def _scatter_kernel(row_hbm, col_hbm, hs_hbm, out_hbm,
                    rowv, colv, bufa, bufb, acc, sema, semb, ssa, ssb):
    c = lax.axis_index("c")
    s = lax.axis_index("s")
    wid = s * NC + c

    @pl.when(jnp.logical_and(c == 0, s < NCH))
    def _init_hs():
        pltpu.sync_copy(hs_hbm.at[pl.ds(s * CH, CH)],
                        acc.at[pl.ds(s * CH, CH)])

    @pl.when(c == 1)
    def _init_zero():
        zeros16 = jnp.zeros((16,), jnp.float32)

        def zstore(t, carry):
            bufa[t // 8, pl.ds((t % 8) * 16, 16)] = zeros16
            return carry

        lax.fori_loop(0, K * D // 16, zstore, 0)

        @pl.when(s < NCH)
        def _fill():
            for t in range(CH // K):
                pltpu.sync_copy(bufa, acc.at[pl.ds(s * CH + t * K, K)])
            pltpu.sync_copy(bufa.at[pl.ds(0, CH - (CH // K) * K)],
                            acc.at[pl.ds(s * CH + (CH // K) * K,
                                         CH - (CH // K) * K)])

    plsc.subcore_barrier()

    for ck in range(NCK):
        pltpu.sync_copy(row_hbm.at[wid, ck], rowv)
        pltpu.sync_copy(col_hbm.at[wid, ck], colv)
        pltpu.async_copy(hs_hbm.at[rowv.at[0]], bufa, sema)
        pltpu.async_copy(hs_hbm.at[rowv.at[1]], bufb, semb)

        def body(i, carry):
            j = i * 2
            pltpu.make_async_copy(hs_hbm.at[rowv.at[j]], bufa, sema).wait()
            pltpu.async_copy(bufa, acc.at[colv.at[j]], ssa, add=True)
            pltpu.make_async_copy(hs_hbm.at[rowv.at[j + 1]], bufb, semb).wait()
            pltpu.async_copy(bufb, acc.at[colv.at[j + 1]], ssb, add=True)
            pltpu.make_async_copy(bufa, acc.at[colv.at[j]], ssa).wait()
            pltpu.async_copy(hs_hbm.at[rowv.at[j + 2]], bufa, sema)
            pltpu.make_async_copy(bufb, acc.at[colv.at[j + 1]], ssb).wait()
            pltpu.async_copy(hs_hbm.at[rowv.at[j + 3]], bufb, semb)
            return carry

        lax.fori_loop(0, (CHB - 2) // 2, body, 0)
        pltpu.make_async_copy(hs_hbm.at[rowv.at[CHB - 2]], bufa, sema).wait()
        pltpu.sync_copy(bufa, acc.at[colv.at[CHB - 2]], add=True)
        pltpu.make_async_copy(hs_hbm.at[rowv.at[CHB - 1]], bufb, semb).wait()
        pltpu.sync_copy(bufb, acc.at[colv.at[CHB - 1]], add=True)

    plsc.subcore_barrier()

    @pl.when(s < NCH)
    def _drain():
        pltpu.sync_copy(acc.at[pl.ds(s * CH, CH)],
                        out_hbm.at[c, pl.ds(s * CH, CH)])


def _mm_body(x_ref, w_ref, o_ref):
    o_ref[...] = jnp.dot(x_ref[...], w_ref[...],
                         preferred_element_type=jnp.float32)


def _scale_body(h_ref, dg_ref, o_ref):
    o_ref[...] = h_ref[...] * lax.rsqrt(dg_ref[...])


def _finish_body(p_ref, dg_ref, b_ref, o_ref):
    acc = p_ref[0] + p_ref[1]
    t = acc * lax.rsqrt(dg_ref[...]) + b_ref[...]
    t = jnp.maximum(t, 0.0)
    m = jnp.max(t, axis=1, keepdims=True)
    e = jnp.exp(t - m)
    o_ref[...] = e / jnp.sum(e, axis=1, keepdims=True)


_RB = 1000
_G = N // _RB


def kernel(x, edge_index, length, dim, W, b):
    npad = EP - E
    row3 = jnp.concatenate(
        [edge_index[0], jnp.zeros((npad,), jnp.int32)]).reshape(NW, NCK, CHB, K)
    col3 = jnp.concatenate(
        [edge_index[1],
         N + (jnp.arange(npad, dtype=jnp.int32) % TR)]).reshape(NW, NCK, CHB, K)
    ridx = jnp.arange(HR, dtype=jnp.int32).reshape(1, HR)
    zrows = jnp.zeros((HR, D), jnp.float32)

    deg2 = _deg_kernel(col3, ridx, zrows)
    deg_n1 = (deg2[0] + deg2[1]).reshape(-1)[:N].reshape(N, 1) + 1.0

    h = pl.pallas_call(
        _mm_body,
        grid=(_G,),
        in_specs=[pl.BlockSpec((_RB, D), lambda i: (i, 0)),
                  pl.BlockSpec((D, D), lambda i: (0, 0))],
        out_specs=pl.BlockSpec((_RB, D), lambda i: (i, 0)),
        out_shape=jax.ShapeDtypeStruct((N, D), jnp.float32),
    )(x, W)

    hs = pl.pallas_call(
        _scale_body,
        grid=(_G,),
        in_specs=[pl.BlockSpec((_RB, D), lambda i: (i, 0)),
                  pl.BlockSpec((_RB, 1), lambda i: (i, 0))],
        out_specs=pl.BlockSpec((_RB, D), lambda i: (i, 0)),
        out_shape=jax.ShapeDtypeStruct((N, D), jnp.float32),
    )(h, deg_n1)

    p = _scatter_kernel(row3, col3, hs)

    out = pl.pallas_call(
        _finish_body,
        grid=(_G,),
        in_specs=[pl.BlockSpec((NC, _RB, D), lambda i: (0, i, 0)),
                  pl.BlockSpec((_RB, 1), lambda i: (i, 0)),
                  pl.BlockSpec((1, D), lambda i: (0, 0))],
        out_specs=pl.BlockSpec((_RB, D), lambda i: (i, 0)),
        out_shape=jax.ShapeDtypeStruct((N, D), jnp.float32),
    )(p, deg_n1, b.reshape(1, D))
    return out


# speedup vs baseline: 1.1862x; 1.1862x over previous
import functools

import jax
import jax.numpy as jnp
from jax import lax
from jax.experimental import pallas as pl
from jax.experimental.pallas import tpu as pltpu
from jax.experimental.pallas import tpu_sc as plsc

N = 10000
E = 320000
D = 128

NC = 2
NS = 16
NW = NC * NS
K = 128
CHB = 8
NCK = 10
NB = NCK * CHB
EPT = NB * K
EP = NW * EPT
TR = 240
CH = 1000
NCH = N // CH

_mesh = plsc.VectorSubcoreMesh(core_axis_name="c", subcore_axis_name="s")


NP = 10240
HR = NP // D


@functools.partial(
    pl.kernel,
    out_type=jax.ShapeDtypeStruct((NC, HR, D), jnp.float32),
    mesh=_mesh,
    compiler_params=pltpu.CompilerParams(needs_layout_passes=False),
    scratch_types=[
        pltpu.VMEM((CHB, K), jnp.int32),
        pltpu.VMEM((HR, D), jnp.float32),
        pltpu.VMEM((1, HR), jnp.int32),
        pltpu.VMEM_SHARED((HR, D), jnp.float32),
    ],
)
def _deg_kernel(col_hbm, ridx_hbm, zrows_hbm, out_hbm, colb, hist, ridx, dacc):
    c = lax.axis_index("c")
    s = lax.axis_index("s")
    wid = s * NC + c

    @pl.when(s == 0)
    def _init():
        pltpu.sync_copy(zrows_hbm, dacc)

    pltpu.sync_copy(zrows_hbm, hist)
    pltpu.sync_copy(ridx_hbm, ridx)
    ones16 = jnp.ones((16,), jnp.float32)

    for ck in range(NCK):
        pltpu.sync_copy(col_hbm.at[wid, ck], colb)

        def body(t, carry):
            r = t // 8
            kk = t % 8
            idx = colb[r, pl.ds(kk * 16, 16)]
            rr = lax.shift_right_logical(idx, 7)
            qq = lax.bitwise_and(idx, 127)
            plsc.addupdate_scatter(hist, [rr, qq], ones16)
            return carry

        lax.fori_loop(0, CHB * K // 16, body, 0)
    plsc.subcore_barrier()
    pltpu.sync_copy(hist, dacc.at[ridx.at[0]], add=True)
    plsc.subcore_barrier()

    @pl.when(s == 0)
    def _drain():
        pltpu.sync_copy(dacc, out_hbm.at[c])


@functools.partial(
    pl.kernel,
    out_type=jax.ShapeDtypeStruct((NC, N, D), jnp.float32),
    mesh=_mesh,
    scratch_types=[
        pltpu.VMEM((CHB, K), jnp.int32),
        pltpu.VMEM((CHB, K), jnp.int32),
        pltpu.VMEM((K, D), jnp.float32),
        pltpu.VMEM((K, D), jnp.float32),
        pltpu.VMEM_SHARED((N + TR, D), jnp.float32),
        pltpu.SemaphoreType.DMA,
        pltpu.SemaphoreType.DMA,
        pltpu.SemaphoreType.DMA,
        pltpu.SemaphoreType.DMA,
    ],
)
def _scatter_kernel(row_hbm, col_hbm, hs_hbm, out_hbm,
                    rowv, colv, bufa, bufb, acc, sema, semb, ssa, ssb):
    c = lax.axis_index("c")
    s = lax.axis_index("s")
    wid = s * NC + c

    @pl.when(jnp.logical_and(c == 0, s < NCH))
    def _init_hs():
        pltpu.sync_copy(hs_hbm.at[pl.ds(s * CH, CH)],
                        acc.at[pl.ds(s * CH, CH)])

    @pl.when(c == 1)
    def _init_zero():
        zeros16 = jnp.zeros((16,), jnp.float32)

        def zstore(t, carry):
            bufa[t // 8, pl.ds((t % 8) * 16, 16)] = zeros16
            return carry

        lax.fori_loop(0, K * D // 16, zstore, 0)

        @pl.when(s < NCH)
        def _fill():
            for t in range(CH // K):
                pltpu.sync_copy(bufa, acc.at[pl.ds(s * CH + t * K, K)])
            pltpu.sync_copy(bufa.at[pl.ds(0, CH - (CH // K) * K)],
                            acc.at[pl.ds(s * CH + (CH // K) * K,
                                         CH - (CH // K) * K)])

    plsc.subcore_barrier()

    for ck in range(NCK):
        pltpu.sync_copy(row_hbm.at[wid, ck], rowv)
        pltpu.sync_copy(col_hbm.at[wid, ck], colv)
        pltpu.async_copy(hs_hbm.at[rowv.at[0]], bufa, sema)
        pltpu.async_copy(hs_hbm.at[rowv.at[1]], bufb, semb)

        def body(i, carry):
            j = i * 2
            pltpu.make_async_copy(hs_hbm.at[rowv.at[j]], bufa, sema).wait()
            pltpu.async_copy(bufa, acc.at[colv.at[j]], ssa, add=True)
            pltpu.make_async_copy(hs_hbm.at[rowv.at[j + 1]], bufb, semb).wait()
            pltpu.async_copy(bufb, acc.at[colv.at[j + 1]], ssb, add=True)
            pltpu.make_async_copy(bufa, acc.at[colv.at[j]], ssa).wait()
            pltpu.async_copy(hs_hbm.at[rowv.at[j + 2]], bufa, sema)
            pltpu.make_async_copy(bufb, acc.at[colv.at[j + 1]], ssb).wait()
            pltpu.async_copy(hs_hbm.at[rowv.at[j + 3]], bufb, semb)
            return carry

        lax.fori_loop(0, (CHB - 2) // 2, body, 0)
        pltpu.make_async_copy(hs_hbm.at[rowv.at[CHB - 2]], bufa, sema).wait()
        pltpu.sync_copy(bufa, acc.at[colv.at[CHB - 2]], add=True)
        pltpu.make_async_copy(hs_hbm.at[rowv.at[CHB - 1]], bufb, semb).wait()
        pltpu.sync_copy(bufb, acc.at[colv.at[CHB - 1]], add=True)

    plsc.subcore_barrier()

    @pl.when(s < NCH)
    def _drain():
        pltpu.sync_copy(acc.at[pl.ds(s * CH, CH)],
                        out_hbm.at[c, pl.ds(s * CH, CH)])


def _mm_body(x_ref, w_ref, o_ref):
    o_ref[...] = jnp.dot(x_ref[...], w_ref[...],
                         preferred_element_type=jnp.float32)


def _scale_body(h_ref, dg_ref, o_ref):
    o_ref[...] = h_ref[...] * lax.rsqrt(dg_ref[...])


def _finish_body(p_ref, dg_ref, b_ref, o_ref):
    acc = p_ref[0] + p_ref[1]
    t = acc * lax.rsqrt(dg_ref[...]) + b_ref[...]
    t = jnp.maximum(t, 0.0)
    m = jnp.max(t, axis=1, keepdims=True)
    e = jnp.exp(t - m)
    o_ref[...] = e / jnp.sum(e, axis=1, keepdims=True)


_RB = 1000
_G = N // _RB


def kernel(x, edge_index, length, dim, W, b):
    ppt = (EP - E) // NW
    row3 = jnp.concatenate(
        [edge_index[0].reshape(NW, E // NW),
         jnp.zeros((NW, ppt), jnp.int32)], axis=1).reshape(NW, NCK, CHB, K)
    colpad = N + (jnp.arange(NW * ppt, dtype=jnp.int32) % TR).reshape(NW, ppt)
    col3 = jnp.concatenate(
        [edge_index[1].reshape(NW, E // NW), colpad],
        axis=1).reshape(NW, NCK, CHB, K)
    ridx = jnp.arange(HR, dtype=jnp.int32).reshape(1, HR)
    zrows = jnp.zeros((HR, D), jnp.float32)

    deg2 = _deg_kernel(col3, ridx, zrows)
    deg_n1 = (deg2[0] + deg2[1]).reshape(-1)[:N].reshape(N, 1) + 1.0

    h = pl.pallas_call(
        _mm_body,
        grid=(_G,),
        in_specs=[pl.BlockSpec((_RB, D), lambda i: (i, 0)),
                  pl.BlockSpec((D, D), lambda i: (0, 0))],
        out_specs=pl.BlockSpec((_RB, D), lambda i: (i, 0)),
        out_shape=jax.ShapeDtypeStruct((N, D), jnp.float32),
    )(x, W)

    hs = pl.pallas_call(
        _scale_body,
        grid=(_G,),
        in_specs=[pl.BlockSpec((_RB, D), lambda i: (i, 0)),
                  pl.BlockSpec((_RB, 1), lambda i: (i, 0))],
        out_specs=pl.BlockSpec((_RB, D), lambda i: (i, 0)),
        out_shape=jax.ShapeDtypeStruct((N, D), jnp.float32),
    )(h, deg_n1)

    p = _scatter_kernel(row3, col3, hs)

    out = pl.pallas_call(
        _finish_body,
        grid=(_G,),
        in_specs=[pl.BlockSpec((NC, _RB, D), lambda i: (0, i, 0)),
                  pl.BlockSpec((_RB, 1), lambda i: (i, 0)),
                  pl.BlockSpec((1, D), lambda i: (0, 0))],
        out_specs=pl.BlockSpec((_RB, D), lambda i: (i, 0)),
        out_shape=jax.ShapeDtypeStruct((N, D), jnp.float32),
    )(p, deg_n1, b.reshape(1, D))
    return out


# speedup vs baseline: 2.6991x; 2.2754x over previous
import functools

import jax
import jax.numpy as jnp
from jax import lax
from jax.experimental import pallas as pl
from jax.experimental.pallas import tpu as pltpu
from jax.experimental.pallas import tpu_sc as plsc

N = 10000
E = 320000
D = 128

NC = 2
NS = 16
NW = NC * NS
K = 80
CHB = 25
NCK = 5
NB = NCK * CHB
EPT = NB * K
CH = 1000
NCH = N // CH

_mesh = plsc.VectorSubcoreMesh(core_axis_name="c", subcore_axis_name="s")


NP = 10240
HR = NP // D


@functools.partial(
    pl.kernel,
    out_type=jax.ShapeDtypeStruct((NC, HR, D), jnp.float32),
    mesh=_mesh,
    compiler_params=pltpu.CompilerParams(needs_layout_passes=False),
    scratch_types=[
        pltpu.VMEM((CHB, K), jnp.int32),
        pltpu.VMEM((HR, D), jnp.float32),
        pltpu.VMEM((1, HR), jnp.int32),
        pltpu.VMEM_SHARED((HR, D), jnp.float32),
    ],
)
def _deg_kernel(ei_hbm, ridx_hbm, zrows_hbm, out_hbm, colb, hist, ridx, dacc):
    c = lax.axis_index("c")
    s = lax.axis_index("s")
    wid = s * NC + c

    @pl.when(s == 0)
    def _init():
        pltpu.sync_copy(zrows_hbm, dacc)

    pltpu.sync_copy(zrows_hbm, hist)
    pltpu.sync_copy(ridx_hbm, ridx)
    ones16 = jnp.ones((16,), jnp.float32)

    for ck in range(NCK):
        pltpu.sync_copy(ei_hbm.at[1, wid, ck], colb)

        def body(t, carry):
            r = t // 5
            kk = t % 5
            idx = colb[r, pl.ds(kk * 16, 16)]
            rr = lax.shift_right_logical(idx, 7)
            qq = lax.bitwise_and(idx, 127)
            plsc.addupdate_scatter(hist, [rr, qq], ones16)
            return carry

        lax.fori_loop(0, CHB * K // 16, body, 0)
    plsc.subcore_barrier()
    pltpu.sync_copy(hist, dacc.at[ridx.at[0]], add=True)
    plsc.subcore_barrier()

    @pl.when(s == 0)
    def _drain():
        pltpu.sync_copy(dacc, out_hbm.at[c])


@functools.partial(
    pl.kernel,
    out_type=jax.ShapeDtypeStruct((NC, N, D), jnp.float32),
    mesh=_mesh,
    scratch_types=[
        pltpu.VMEM((CHB, K), jnp.int32),
        pltpu.VMEM((CHB, K), jnp.int32),
        pltpu.VMEM((K, D), jnp.float32),
        pltpu.VMEM((K, D), jnp.float32),
        pltpu.VMEM_SHARED((N, D), jnp.float32),
        pltpu.SemaphoreType.DMA,
        pltpu.SemaphoreType.DMA,
        pltpu.SemaphoreType.DMA,
        pltpu.SemaphoreType.DMA,
    ],
)
def _scatter_kernel(ei_hbm, hs_hbm, out_hbm,
                    rowv, colv, bufa, bufb, acc, sema, semb, ssa, ssb):
    c = lax.axis_index("c")
    s = lax.axis_index("s")
    wid = s * NC + c

    @pl.when(jnp.logical_and(c == 0, s < NCH))
    def _init_hs():
        pltpu.sync_copy(hs_hbm.at[pl.ds(s * CH, CH)],
                        acc.at[pl.ds(s * CH, CH)])

    @pl.when(c == 1)
    def _init_zero():
        zeros16 = jnp.zeros((16,), jnp.float32)

        def zstore(t, carry):
            bufa[t // 8, pl.ds((t % 8) * 16, 16)] = zeros16
            return carry

        lax.fori_loop(0, K * D // 16, zstore, 0)

        @pl.when(s < NCH)
        def _fill():
            for t in range(CH // K):
                pltpu.sync_copy(bufa, acc.at[pl.ds(s * CH + t * K, K)])
            pltpu.sync_copy(bufa.at[pl.ds(0, CH - (CH // K) * K)],
                            acc.at[pl.ds(s * CH + (CH // K) * K,
                                         CH - (CH // K) * K)])

    plsc.subcore_barrier()

    for ck in range(NCK):
        pltpu.sync_copy(ei_hbm.at[0, wid, ck], rowv)
        pltpu.sync_copy(ei_hbm.at[1, wid, ck], colv)
        pltpu.async_copy(hs_hbm.at[rowv.at[0]], bufa, sema)
        pltpu.async_copy(hs_hbm.at[rowv.at[1]], bufb, semb)

        def body(i, carry):
            j = i * 2
            pltpu.make_async_copy(hs_hbm.at[rowv.at[j]], bufa, sema).wait()
            pltpu.async_copy(bufa, acc.at[colv.at[j]], ssa, add=True)
            pltpu.make_async_copy(hs_hbm.at[rowv.at[j + 1]], bufb, semb).wait()
            pltpu.async_copy(bufb, acc.at[colv.at[j + 1]], ssb, add=True)
            pltpu.make_async_copy(bufa, acc.at[colv.at[j]], ssa).wait()
            pltpu.async_copy(hs_hbm.at[rowv.at[j + 2]], bufa, sema)
            pltpu.make_async_copy(bufb, acc.at[colv.at[j + 1]], ssb).wait()
            pltpu.async_copy(hs_hbm.at[rowv.at[j + 3]], bufb, semb)
            return carry

        lax.fori_loop(0, (CHB - 3) // 2, body, 0)
        pltpu.make_async_copy(hs_hbm.at[rowv.at[CHB - 3]], bufa, sema).wait()
        pltpu.sync_copy(bufa, acc.at[colv.at[CHB - 3]], add=True)
        pltpu.make_async_copy(hs_hbm.at[rowv.at[CHB - 2]], bufb, semb).wait()
        pltpu.sync_copy(bufb, acc.at[colv.at[CHB - 2]], add=True)
        pltpu.async_copy(hs_hbm.at[rowv.at[CHB - 1]], bufa, sema).wait()
        pltpu.sync_copy(bufa, acc.at[colv.at[CHB - 1]], add=True)

    plsc.subcore_barrier()

    @pl.when(s < NCH)
    def _drain():
        pltpu.sync_copy(acc.at[pl.ds(s * CH, CH)],
                        out_hbm.at[c, pl.ds(s * CH, CH)])


def _mm_body(x_ref, w_ref, o_ref):
    o_ref[...] = jnp.dot(x_ref[...], w_ref[...],
                         preferred_element_type=jnp.float32)


def _scale_body(h_ref, dg_ref, o_ref):
    o_ref[...] = h_ref[...] * lax.rsqrt(dg_ref[...])


def _finish_body(p_ref, dg_ref, b_ref, o_ref):
    acc = p_ref[0] + p_ref[1]
    t = acc * lax.rsqrt(dg_ref[...]) + b_ref[...]
    t = jnp.maximum(t, 0.0)
    m = jnp.max(t, axis=1, keepdims=True)
    e = jnp.exp(t - m)
    o_ref[...] = e / jnp.sum(e, axis=1, keepdims=True)


_RB = 1000
_G = N // _RB


def kernel(x, edge_index, length, dim, W, b):
    ei5 = edge_index.reshape(2, NW, NCK, CHB, K)
    ridx = jnp.arange(HR, dtype=jnp.int32).reshape(1, HR)
    zrows = jnp.zeros((HR, D), jnp.float32)

    deg2 = _deg_kernel(ei5, ridx, zrows)
    deg_n1 = (deg2[0] + deg2[1]).reshape(-1)[:N].reshape(N, 1) + 1.0

    h = pl.pallas_call(
        _mm_body,
        grid=(_G,),
        in_specs=[pl.BlockSpec((_RB, D), lambda i: (i, 0)),
                  pl.BlockSpec((D, D), lambda i: (0, 0))],
        out_specs=pl.BlockSpec((_RB, D), lambda i: (i, 0)),
        out_shape=jax.ShapeDtypeStruct((N, D), jnp.float32),
    )(x, W)

    hs = pl.pallas_call(
        _scale_body,
        grid=(_G,),
        in_specs=[pl.BlockSpec((_RB, D), lambda i: (i, 0)),
                  pl.BlockSpec((_RB, 1), lambda i: (i, 0))],
        out_specs=pl.BlockSpec((_RB, D), lambda i: (i, 0)),
        out_shape=jax.ShapeDtypeStruct((N, D), jnp.float32),
    )(h, deg_n1)

    p = _scatter_kernel(ei5, hs)

    out = pl.pallas_call(
        _finish_body,
        grid=(_G,),
        in_specs=[pl.BlockSpec((NC, _RB, D), lambda i: (0, i, 0)),
                  pl.BlockSpec((_RB, 1), lambda i: (i, 0)),
                  pl.BlockSpec((1, D), lambda i: (0, 0))],
        out_specs=pl.BlockSpec((_RB, D), lambda i: (i, 0)),
        out_shape=jax.ShapeDtypeStruct((N, D), jnp.float32),
    )(p, deg_n1, b.reshape(1, D))
    return out


# speedup vs baseline: 3.0143x; 1.1168x over previous
import functools

import jax
import jax.numpy as jnp
from jax import lax
from jax.experimental import pallas as pl
from jax.experimental.pallas import tpu as pltpu
from jax.experimental.pallas import tpu_sc as plsc

N = 10000
E = 320000
D = 128

NC = 2
NS = 16
NW = NC * NS
K = 80
CHB = 25
NCK = 5
NB = NCK * CHB
EPT = NB * K
CH = 1000
NCH = N // CH

_mesh = plsc.VectorSubcoreMesh(core_axis_name="c", subcore_axis_name="s")


NP = 10240
HR = NP // D


@functools.partial(
    pl.kernel,
    out_type=jax.ShapeDtypeStruct((NC, HR, D), jnp.float32),
    mesh=_mesh,
    compiler_params=pltpu.CompilerParams(needs_layout_passes=False),
    scratch_types=[
        pltpu.VMEM((CHB, K), jnp.int32),
        pltpu.VMEM((HR, D), jnp.float32),
        pltpu.VMEM((1, HR), jnp.int32),
        pltpu.VMEM_SHARED((HR, D), jnp.float32),
    ],
)
def _deg_kernel(ei_hbm, ridx_hbm, zrows_hbm, out_hbm, colb, hist, ridx, dacc):
    c = lax.axis_index("c")
    s = lax.axis_index("s")
    wid = s * NC + c

    @pl.when(s == 0)
    def _init():
        pltpu.sync_copy(zrows_hbm, dacc)

    pltpu.sync_copy(zrows_hbm, hist)
    pltpu.sync_copy(ridx_hbm, ridx)
    ones16 = jnp.ones((16,), jnp.float32)

    for ck in range(NCK):
        pltpu.sync_copy(ei_hbm.at[1, wid, ck], colb)

        def body(t, carry):
            r = t // 5
            kk = t % 5
            idx = colb[r, pl.ds(kk * 16, 16)]
            rr = lax.shift_right_logical(idx, 7)
            qq = lax.bitwise_and(idx, 127)
            plsc.addupdate_scatter(hist, [rr, qq], ones16)
            return carry

        lax.fori_loop(0, CHB * K // 16, body, 0)
    plsc.subcore_barrier()
    pltpu.sync_copy(hist, dacc.at[ridx.at[0]], add=True)
    plsc.subcore_barrier()

    @pl.when(s == 0)
    def _drain():
        pltpu.sync_copy(dacc, out_hbm.at[c])


@functools.partial(
    pl.kernel,
    out_type=jax.ShapeDtypeStruct((NC, N, D), jnp.float32),
    mesh=_mesh,
    scratch_types=[
        pltpu.VMEM((CHB, K), jnp.int32),
        pltpu.VMEM((CHB, K), jnp.int32),
        pltpu.VMEM((K, D), jnp.float32),
        pltpu.VMEM((K, D), jnp.float32),
        pltpu.VMEM((K, D), jnp.float32),
        pltpu.VMEM_SHARED((N, D), jnp.float32),
        pltpu.SemaphoreType.DMA,
        pltpu.SemaphoreType.DMA,
        pltpu.SemaphoreType.DMA,
        pltpu.SemaphoreType.DMA,
        pltpu.SemaphoreType.DMA,
        pltpu.SemaphoreType.DMA,
    ],
)
def _scatter_kernel(ei_hbm, hs_hbm, out_hbm,
                    rowv, colv, bufa, bufb, bufc, acc,
                    sema, semb, semc, ssa, ssb, ssc):
    c = lax.axis_index("c")
    s = lax.axis_index("s")
    wid = s * NC + c

    @pl.when(jnp.logical_and(c == 0, s < NCH))
    def _init_hs():
        pltpu.sync_copy(hs_hbm.at[pl.ds(s * CH, CH)],
                        acc.at[pl.ds(s * CH, CH)])

    @pl.when(c == 1)
    def _init_zero():
        zeros16 = jnp.zeros((16,), jnp.float32)

        def zstore(t, carry):
            bufa[t // 8, pl.ds((t % 8) * 16, 16)] = zeros16
            return carry

        lax.fori_loop(0, K * D // 16, zstore, 0)

        @pl.when(s < NCH)
        def _fill():
            for t in range(CH // K):
                pltpu.sync_copy(bufa, acc.at[pl.ds(s * CH + t * K, K)])
            pltpu.sync_copy(bufa.at[pl.ds(0, CH - (CH // K) * K)],
                            acc.at[pl.ds(s * CH + (CH // K) * K,
                                         CH - (CH // K) * K)])

    plsc.subcore_barrier()

    for ck in range(NCK):
        pltpu.sync_copy(ei_hbm.at[0, wid, ck], rowv)
        pltpu.sync_copy(ei_hbm.at[1, wid, ck], colv)
        pltpu.async_copy(hs_hbm.at[rowv.at[0]], bufa, sema)
        pltpu.async_copy(hs_hbm.at[rowv.at[1]], bufb, semb)
        pltpu.async_copy(hs_hbm.at[rowv.at[2]], bufc, semc)

        def body(i, carry):
            j = i * 3
            pltpu.make_async_copy(hs_hbm.at[rowv.at[j]], bufa, sema).wait()
            pltpu.async_copy(bufa, acc.at[colv.at[j]], ssa, add=True)
            pltpu.make_async_copy(hs_hbm.at[rowv.at[j + 1]], bufb, semb).wait()
            pltpu.async_copy(bufb, acc.at[colv.at[j + 1]], ssb, add=True)
            pltpu.make_async_copy(hs_hbm.at[rowv.at[j + 2]], bufc, semc).wait()
            pltpu.async_copy(bufc, acc.at[colv.at[j + 2]], ssc, add=True)
            pltpu.make_async_copy(bufa, acc.at[colv.at[j]], ssa).wait()
            pltpu.async_copy(hs_hbm.at[rowv.at[j + 3]], bufa, sema)
            pltpu.make_async_copy(bufb, acc.at[colv.at[j + 1]], ssb).wait()
            pltpu.async_copy(hs_hbm.at[rowv.at[j + 4]], bufb, semb)
            pltpu.make_async_copy(bufc, acc.at[colv.at[j + 2]], ssc).wait()
            pltpu.async_copy(hs_hbm.at[rowv.at[j + 5]], bufc, semc)
            return carry

        lax.fori_loop(0, (CHB - 4) // 3, body, 0)
        pltpu.make_async_copy(hs_hbm.at[rowv.at[CHB - 4]], bufa, sema).wait()
        pltpu.sync_copy(bufa, acc.at[colv.at[CHB - 4]], add=True)
        pltpu.make_async_copy(hs_hbm.at[rowv.at[CHB - 3]], bufb, semb).wait()
        pltpu.sync_copy(bufb, acc.at[colv.at[CHB - 3]], add=True)
        pltpu.make_async_copy(hs_hbm.at[rowv.at[CHB - 2]], bufc, semc).wait()
        pltpu.sync_copy(bufc, acc.at[colv.at[CHB - 2]], add=True)
        pltpu.async_copy(hs_hbm.at[rowv.at[CHB - 1]], bufa, sema).wait()
        pltpu.sync_copy(bufa, acc.at[colv.at[CHB - 1]], add=True)

    plsc.subcore_barrier()

    @pl.when(s < NCH)
    def _drain():
        pltpu.sync_copy(acc.at[pl.ds(s * CH, CH)],
                        out_hbm.at[c, pl.ds(s * CH, CH)])


def _mm_body(x_ref, w_ref, o_ref):
    o_ref[...] = jnp.dot(x_ref[...], w_ref[...],
                         preferred_element_type=jnp.float32)


def _scale_body(h_ref, dg_ref, o_ref):
    o_ref[...] = h_ref[...] * lax.rsqrt(dg_ref[...])


def _finish_body(p_ref, dg_ref, b_ref, o_ref):
    acc = p_ref[0] + p_ref[1]
    t = acc * lax.rsqrt(dg_ref[...]) + b_ref[...]
    t = jnp.maximum(t, 0.0)
    m = jnp.max(t, axis=1, keepdims=True)
    e = jnp.exp(t - m)
    o_ref[...] = e / jnp.sum(e, axis=1, keepdims=True)


_RB = 1000
_G = N // _RB


def kernel(x, edge_index, length, dim, W, b):
    ei5 = edge_index.reshape(2, NW, NCK, CHB, K)
    ridx = jnp.arange(HR, dtype=jnp.int32).reshape(1, HR)
    zrows = jnp.zeros((HR, D), jnp.float32)

    deg2 = _deg_kernel(ei5, ridx, zrows)
    deg_n1 = (deg2[0] + deg2[1]).reshape(-1)[:N].reshape(N, 1) + 1.0

    h = pl.pallas_call(
        _mm_body,
        grid=(_G,),
        in_specs=[pl.BlockSpec((_RB, D), lambda i: (i, 0)),
                  pl.BlockSpec((D, D), lambda i: (0, 0))],
        out_specs=pl.BlockSpec((_RB, D), lambda i: (i, 0)),
        out_shape=jax.ShapeDtypeStruct((N, D), jnp.float32),
    )(x, W)

    hs = pl.pallas_call(
        _scale_body,
        grid=(_G,),
        in_specs=[pl.BlockSpec((_RB, D), lambda i: (i, 0)),
                  pl.BlockSpec((_RB, 1), lambda i: (i, 0))],
        out_specs=pl.BlockSpec((_RB, D), lambda i: (i, 0)),
        out_shape=jax.ShapeDtypeStruct((N, D), jnp.float32),
    )(h, deg_n1)

    p = _scatter_kernel(ei5, hs)

    out = pl.pallas_call(
        _finish_body,
        grid=(_G,),
        in_specs=[pl.BlockSpec((NC, _RB, D), lambda i: (0, i, 0)),
                  pl.BlockSpec((_RB, 1), lambda i: (i, 0)),
                  pl.BlockSpec((1, D), lambda i: (0, 0))],
        out_specs=pl.BlockSpec((_RB, D), lambda i: (i, 0)),
        out_shape=jax.ShapeDtypeStruct((N, D), jnp.float32),
    )(p, deg_n1, b.reshape(1, D))
    return out
